# Initial kernel scaffold; baseline (speedup 1.0000x reference)
import jax
import jax.numpy as jnp
from jax.experimental import pallas as pl


def kernel(x, edge_index, edge_attr, batch, We, be, linW, linb, mlpW, mlpb, n2W, n2b, W1, b1, W2, b2, W3, b3):
    raise NotImplementedError("write your pallas kernel here")


# speedup vs baseline: 2.1809x; 2.1809x over previous
import functools

import jax
import jax.numpy as jnp
from jax import lax
from jax.experimental import pallas as pl
from jax.experimental.pallas import tpu as pltpu
from jax.experimental.pallas import tpu_sc as plsc

_N = 10000
_E = 160000
_HP = 304
_G = 64
_CH = 128
_NCHUNK = _E // _CH
_NW = 32
_HALF = _N // 2
_ZR = 320
_ROWS = _ZR * 16


def _sc_gather(table, idx, d):
    per_w = (_NCHUNK + _NW - 1) // _NW
    mesh = plsc.VectorSubcoreMesh(core_axis_name="c", subcore_axis_name="s")

    @functools.partial(
        pl.kernel,
        out_type=jax.ShapeDtypeStruct((_E, d), jnp.float32),
        mesh=mesh,
        scratch_types=[
            pltpu.VMEM((_CH,), jnp.int32),
            pltpu.VMEM((_CH, d), jnp.float32),
            pltpu.SemaphoreType.DMA,
        ],
        compiler_params=pltpu.CompilerParams(use_tc_tiling_on_sc=False),
    )
    def k(table_hbm, idx_hbm, out_hbm, idx_v, rows_v, sem):
        c = lax.axis_index("c")
        s = lax.axis_index("s")
        wid = s * 2 + c

        def body(j, carry):
            cid = wid + j * _NW

            @pl.when(cid < _NCHUNK)
            def _():
                base = cid * _CH
                pltpu.sync_copy(idx_hbm.at[pl.ds(base, _CH)], idx_v)
                pltpu.async_copy(table_hbm.at[idx_v], rows_v, sem).wait()
                pltpu.sync_copy(rows_v, out_hbm.at[pl.ds(base, _CH)])

            return carry

        lax.fori_loop(0, per_w, body, 0)

    return k(table, idx)


_HC = _HP // 2


def _sc_scatter_add(msg, idx, zrows):
    per_t = (_NCHUNK + 15) // 16
    mesh = plsc.VectorSubcoreMesh(core_axis_name="c", subcore_axis_name="s")

    @functools.partial(
        pl.kernel,
        out_type=jax.ShapeDtypeStruct((_N, _HP), jnp.float32),
        mesh=mesh,
        scratch_types=[
            pltpu.VMEM((_CH,), jnp.int32),
            pltpu.VMEM((_CH, _HC), jnp.float32),
            pltpu.VMEM_SHARED((_ROWS, _HC), jnp.float32),
        ],
        compiler_params=pltpu.CompilerParams(use_tc_tiling_on_sc=False),
    )
    def k(msg_hbm, idx_hbm, z_hbm, out_hbm, idx_v, msg_v, acc):
        c = lax.axis_index("c")
        s = lax.axis_index("s")
        base_node = c * _HALF

        for p in range(2):
            pltpu.sync_copy(z_hbm, acc.at[pl.ds(s * _ZR, _ZR)])
            plsc.subcore_barrier()

            def body(j, carry):
                cid = s + j * 16

                @pl.when(cid < _NCHUNK)
                def _():
                    base = cid * _CH
                    pltpu.sync_copy(idx_hbm.at[pl.ds(base, _CH)], idx_v)
                    pltpu.sync_copy(
                        msg_hbm.at[pl.ds(base, _CH), pl.ds(p * _HC, _HC)],
                        msg_v)
                    for q in range(_CH // 16):
                        v = idx_v[pl.ds(q * 16, 16)]
                        li = v - base_node
                        inb = (li >= 0) & (li < _HALF)
                        idx_v[pl.ds(q * 16, 16)] = jnp.where(inb, li, _HALF)
                    pltpu.sync_copy(msg_v, acc.at[idx_v], add=True)

                return carry

            lax.fori_loop(0, per_t, body, 0)
            plsc.subcore_barrier()

            @pl.when(s < 15)
            def _():
                pltpu.sync_copy(
                    acc.at[pl.ds(s * _ZR, _ZR)],
                    out_hbm.at[pl.ds(base_node + s * _ZR, _ZR),
                               pl.ds(p * _HC, _HC)],
                )

            @pl.when(s == 15)
            def _():
                tail = _HALF - 15 * _ZR
                pltpu.sync_copy(
                    acc.at[pl.ds(15 * _ZR, tail)],
                    out_hbm.at[pl.ds(base_node + 15 * _ZR, tail),
                               pl.ds(p * _HC, _HC)],
                )

            plsc.subcore_barrier()

    return k(msg, idx, zrows)


_BE = 640
_GRID_E = _E // _BE


def _silu(x):
    return x * jax.nn.sigmoid(x)


def _tc_edge_init(xg, ea, WeX, WeE, be2, W0, b0):
    def body(xg_ref, ea_ref, wx_ref, we_ref, be_ref, w0_ref, b0_ref,
             e_ref, m_ref):
        h = jnp.dot(xg_ref[...], wx_ref[...], preferred_element_type=jnp.float32)
        h = h + jnp.dot(ea_ref[...], we_ref[...], preferred_element_type=jnp.float32)
        h = h + be_ref[...]
        e = _silu(h)
        e_ref[...] = e
        m = jnp.dot(e, w0_ref[...], preferred_element_type=jnp.float32) + b0_ref[...]
        m_ref[...] = jnp.maximum(m, 0.0)

    full = lambda shape: pl.BlockSpec(shape, lambda i: (0, 0))
    return pl.pallas_call(
        body,
        grid=(_GRID_E,),
        in_specs=[
            pl.BlockSpec((_BE, 128), lambda i: (i, 0)),
            pl.BlockSpec((_BE, 16), lambda i: (i, 0)),
            full((128, _HP)),
            full((16, _HP)),
            full((1, _HP)),
            full((_HP, _HP)),
            full((1, _HP)),
        ],
        out_specs=[
            pl.BlockSpec((_BE, _HP), lambda i: (i, 0)),
            pl.BlockSpec((_BE, _HP), lambda i: (i, 0)),
        ],
        out_shape=[
            jax.ShapeDtypeStruct((_E, _HP), jnp.float32),
            jax.ShapeDtypeStruct((_E, _HP), jnp.float32),
        ],
        compiler_params=pltpu.CompilerParams(
            dimension_semantics=("parallel",)),
    )(xg, ea, WeX, WeE, be2, W0, b0)


def _tc_layer(ag, ep, mW, mb2, nW, nb2, last):
    def body(ag_ref, ep_ref, mw_ref, mb_ref, nw_ref, nb_ref, *outs):
        d = ag_ref[...] - ep_ref[...]
        h = jnp.dot(d, mw_ref[...], preferred_element_type=jnp.float32) + mb_ref[...]
        h = jnp.maximum(h, 0.0)
        if last:
            e = 2.0 * h
            m_ref, = outs
        else:
            e = _silu(h) + h
            e_ref, m_ref = outs
            e_ref[...] = e
        m = jnp.dot(e, nw_ref[...], preferred_element_type=jnp.float32) + nb_ref[...]
        m_ref[...] = jnp.maximum(m, 0.0)

    full = lambda shape: pl.BlockSpec(shape, lambda i: (0, 0))
    n_out = 1 if last else 2
    return pl.pallas_call(
        body,
        grid=(_GRID_E,),
        in_specs=[
            pl.BlockSpec((_BE, _HP), lambda i: (i, 0)),
            pl.BlockSpec((_BE, _HP), lambda i: (i, 0)),
            full((_HP, _HP)),
            full((1, _HP)),
            full((_HP, _HP)),
            full((1, _HP)),
        ],
        out_specs=[pl.BlockSpec((_BE, _HP), lambda i: (i, 0))] * n_out,
        out_shape=[jax.ShapeDtypeStruct((_E, _HP), jnp.float32)] * n_out,
        compiler_params=pltpu.CompilerParams(
            dimension_semantics=("parallel",)),
    )(ag, ep, mW, mb2, nW, nb2)


_BN = 400
_GRID_N = _N // _BN


def _tc_pool_ffn(node_h, batch3, W1p, b12, W2, b22, W3, b32):
    def body(nh_ref, b_ref, w1_ref, b1_ref, w2_ref, b2_ref, w3_ref, b3_ref,
             out_ref, acc):
        i = pl.program_id(0)
        seg = jnp.broadcast_to(b_ref[0], (_G, _BN))
        gids = lax.broadcasted_iota(jnp.int32, (_G, _BN), 0)
        onehot = (seg == gids).astype(jnp.float32)
        part = jnp.dot(onehot, nh_ref[...], preferred_element_type=jnp.float32)

        @pl.when(i == 0)
        def _():
            acc[...] = jnp.zeros_like(acc)

        acc[...] += part

        @pl.when(i == _GRID_N - 1)
        def _():
            h = jnp.dot(acc[...], w1_ref[...], preferred_element_type=jnp.float32)
            h = _silu(h + b1_ref[...])
            h = jnp.dot(h, w2_ref[...], preferred_element_type=jnp.float32)
            h = _silu(h + b2_ref[...])
            h = jnp.dot(h, w3_ref[...], preferred_element_type=jnp.float32)
            out_ref[...] = h + b3_ref[...]

    full = lambda shape: pl.BlockSpec(shape, lambda i: tuple(0 for _ in shape))
    return pl.pallas_call(
        body,
        grid=(_GRID_N,),
        in_specs=[
            pl.BlockSpec((_BN, _HP), lambda i: (i, 0)),
            pl.BlockSpec((1, 1, _BN), lambda i: (i, 0, 0)),
            full((_HP, 300)),
            full((1, 300)),
            full((300, 300)),
            full((1, 300)),
            full((300, 1)),
            full((1, 1)),
        ],
        out_specs=pl.BlockSpec((_G, 1), lambda i: (0, 0)),
        out_shape=jax.ShapeDtypeStruct((_G, 1), jnp.float32),
        scratch_shapes=[pltpu.VMEM((_G, _HP), jnp.float32)],
        compiler_params=pltpu.CompilerParams(
            dimension_semantics=("arbitrary",)),
    )(node_h, batch3, W1p, b12, W2, b22, W3, b32)


def kernel(x, edge_index, edge_attr, batch, We, be, linW, linb, mlpW, mlpb,
           n2W, n2b, W1, b1, W2, b2, W3, b3):
    f32 = jnp.float32
    row = edge_index[0].astype(jnp.int32)
    col = edge_index[1].astype(jnp.int32)
    row_sw = row.reshape(_E // 2, 2)[:, ::-1].reshape(_E)
    col_sw = col.reshape(_E // 2, 2)[:, ::-1].reshape(_E)

    pad_h = lambda w: jnp.pad(w, ((0, _HP - w.shape[0]), (0, _HP - w.shape[1])))
    pad_b = lambda v: jnp.pad(v, (0, _HP - v.shape[0])).reshape(1, _HP)

    WeX = jnp.pad(We[:128], ((0, 0), (0, _HP - 300)))
    WeE = jnp.pad(We[128:], ((0, 0), (0, _HP - 300)))
    be2 = pad_b(be)
    linWp = [pad_h(linW[l]) for l in range(3)]
    linbp = [pad_b(linb[l]) for l in range(3)]
    mlpWp = [pad_h(mlpW[l]) for l in range(3)]
    mlpbp = [pad_b(mlpb[l]) for l in range(3)]
    n2Wp = pad_h(n2W)
    n2bp = pad_b(n2b)
    W1p = jnp.pad(W1, ((0, _HP - 300), (0, 0)))
    b12 = b1.reshape(1, 300)
    b22 = b2.reshape(1, 300)
    b32 = b3.reshape(1, 1)
    batch3 = batch.astype(jnp.int32).reshape(_GRID_N, 1, _BN)
    zrows = jnp.zeros((_ZR, _HC), f32)

    xg = _sc_gather(x, row, 128)
    e0, msg0 = _tc_edge_init(xg, edge_attr, WeX, WeE, be2, linWp[0], linbp[0])

    a0 = _sc_scatter_add(msg0, col, zrows)
    ag0 = _sc_gather(a0, row_sw, _HP)
    e1s, msg1s = _tc_layer(ag0, e0, mlpWp[0], mlpbp[0], linWp[1], linbp[1], False)

    a1 = _sc_scatter_add(msg1s, col_sw, zrows)
    ag1 = _sc_gather(a1, row, _HP)
    e2, msg2 = _tc_layer(ag1, e1s, mlpWp[1], mlpbp[1], linWp[2], linbp[2], False)

    a2 = _sc_scatter_add(msg2, col, zrows)
    ag2 = _sc_gather(a2, row_sw, _HP)
    msgf, = _tc_layer(ag2, e2, mlpWp[2], mlpbp[2], n2Wp, n2bp, True)

    node_h = _sc_scatter_add(msgf, col_sw, zrows)
    return _tc_pool_ffn(node_h, batch3, W1p, b12, W2, b22, W3, b32)


# speedup vs baseline: 2.2389x; 1.0266x over previous
import functools

import jax
import jax.numpy as jnp
from jax import lax
from jax.experimental import pallas as pl
from jax.experimental.pallas import tpu as pltpu
from jax.experimental.pallas import tpu_sc as plsc

_N = 10000
_E = 160000
_HP = 304
_HC = _HP // 2
_G = 64
_CH = 128
_NCHUNK = _E // _CH
_SCH = 64
_NCHUNK_S = _E // _SCH
_NW = 32
_ZSTR = 624
_ZCH = 640
_FCH = 632


def _sc_gather(tables, idx2d, widths):
    nt = len(tables)
    per_w = _NCHUNK // _NW
    rem = _NCHUNK - per_w * _NW
    mesh = plsc.VectorSubcoreMesh(core_axis_name="c", subcore_axis_name="s")

    scratch = [pltpu.VMEM((per_w + 1, _CH), jnp.int32)]
    for t in range(nt):
        for b in range(2):
            scratch.append(pltpu.VMEM((_CH, widths[t]), jnp.float32))
    scratch.append(pltpu.SemaphoreType.DMA)
    scratch += [pltpu.SemaphoreType.DMA, pltpu.SemaphoreType.DMA]

    @functools.partial(
        pl.kernel,
        out_type=[jax.ShapeDtypeStruct((_E, w), jnp.float32) for w in widths],
        mesh=mesh,
        scratch_types=scratch,
        compiler_params=pltpu.CompilerParams(use_tc_tiling_on_sc=False),
    )
    def k(*refs):
        tabs = refs[:nt]
        idx_hbm = refs[nt]
        outs = refs[nt + 1:nt + 1 + nt]
        idx_v = refs[nt + 1 + nt]
        bufs = [[refs[nt + 2 + nt + 2 * t + b] for b in range(2)]
                for t in range(nt)]
        gsem = refs[-3]
        wsem = [refs[-2], refs[-1]]

        c = lax.axis_index("c")
        s = lax.axis_index("s")
        w = s * 2 + c
        start = per_w * w + jnp.minimum(w, rem)
        count = per_w + (w < rem).astype(jnp.int32)

        pltpu.sync_copy(idx_hbm.at[pl.ds(start, per_w)],
                        idx_v.at[pl.ds(0, per_w)])

        @pl.when(w < rem)
        def _():
            pltpu.sync_copy(idx_hbm.at[pl.ds(start + per_w, 1)],
                            idx_v.at[pl.ds(per_w, 1)])

        def chunk(k_, b):
            @pl.when(k_ < count)
            def _():
                @pl.when(k_ >= 2)
                def _():
                    for t in range(nt):
                        pltpu.make_async_copy(
                            bufs[t][b], outs[t].at[pl.ds(0, _CH)],
                            wsem[b]).wait()

                row = idx_v.at[k_]
                descs = [
                    pltpu.async_copy(tabs[t].at[row], bufs[t][b], gsem)
                    for t in range(nt)
                ]
                for dsc in descs:
                    dsc.wait()
                base = (start + k_) * _CH
                for t in range(nt):
                    pltpu.async_copy(bufs[t][b],
                                     outs[t].at[pl.ds(base, _CH)], wsem[b])

        def outer(k2, carry):
            chunk(k2 * 2, 0)
            chunk(k2 * 2 + 1, 1)
            return carry

        lax.fori_loop(0, (per_w + 2) // 2, outer, 0)

        for b in range(2):
            for t in range(nt):
                pltpu.make_async_copy(bufs[t][b], outs[t].at[pl.ds(0, _CH)],
                                      wsem[b]).wait()

    return k(*tables, idx2d)


def _sc_scatter_add(msg2, idx2d, zrows):
    per_t = _NCHUNK_S // 16
    rem = _NCHUNK_S - per_t * 16
    mesh = plsc.VectorSubcoreMesh(core_axis_name="c", subcore_axis_name="s")

    @functools.partial(
        pl.kernel,
        out_type=[jax.ShapeDtypeStruct((_N, _HC), jnp.float32),
                  jax.ShapeDtypeStruct((_N, _HC), jnp.float32)],
        mesh=mesh,
        scratch_types=[
            pltpu.VMEM((per_t + 1, _SCH), jnp.int32),
            pltpu.VMEM((_SCH, _HC), jnp.float32),
            pltpu.VMEM((_SCH, _HC), jnp.float32),
            pltpu.VMEM_SHARED((_N, _HC), jnp.float32),
            pltpu.SemaphoreType.DMA,
            pltpu.SemaphoreType.DMA,
            pltpu.SemaphoreType.DMA,
            pltpu.SemaphoreType.DMA,
        ],
        compiler_params=pltpu.CompilerParams(use_tc_tiling_on_sc=False),
    )
    def k(msg_hbm, idx_hbm, z_hbm, outL, outR, idx_v, mv0, mv1, acc,
          l0, l1, a0, a1):
        c = lax.axis_index("c")
        s = lax.axis_index("s")
        mv = [mv0, mv1]
        lsem = [l0, l1]
        asem = [a0, a1]
        start = per_t * s + jnp.minimum(s, rem)
        count = per_t + (s < rem).astype(jnp.int32)

        pltpu.sync_copy(z_hbm, acc.at[pl.ds(s * _ZSTR, _ZCH)])

        pltpu.sync_copy(idx_hbm.at[pl.ds(start, per_t)],
                        idx_v.at[pl.ds(0, per_t)])

        @pl.when(s < rem)
        def _():
            pltpu.sync_copy(idx_hbm.at[pl.ds(start + per_t, 1)],
                            idx_v.at[pl.ds(per_t, 1)])

        plsc.subcore_barrier()

        def chunk(k_, b):
            @pl.when(k_ < count)
            def _():
                @pl.when(k_ >= 2)
                def _():
                    pltpu.make_async_copy(mv[b], acc.at[pl.ds(0, _SCH)],
                                          asem[b]).wait()
                pltpu.async_copy(
                    msg_hbm.at[c, pl.ds((start + k_) * _SCH, _SCH)],
                    mv[b], lsem[b])
                pltpu.make_async_copy(
                    msg_hbm.at[c, pl.ds(0, _SCH)], mv[b], lsem[b]).wait()
                pltpu.async_copy(mv[b], acc.at[idx_v.at[k_]], asem[b],
                                 add=True)

        def outer(k2, carry):
            chunk(k2 * 2, 0)
            chunk(k2 * 2 + 1, 1)
            return carry

        lax.fori_loop(0, (per_t + 2) // 2, outer, 0)

        for b in range(2):
            pltpu.make_async_copy(mv[b], acc.at[pl.ds(0, _SCH)],
                                  asem[b]).wait()

        plsc.subcore_barrier()

        @pl.when(c == 0)
        def _():
            @pl.when(s < 15)
            def _():
                pltpu.sync_copy(acc.at[pl.ds(s * _FCH, _FCH)],
                                outL.at[pl.ds(s * _FCH, _FCH)])

            @pl.when(s == 15)
            def _():
                pltpu.sync_copy(acc.at[pl.ds(15 * _FCH, _N - 15 * _FCH)],
                                outL.at[pl.ds(15 * _FCH, _N - 15 * _FCH)])

        @pl.when(c == 1)
        def _():
            @pl.when(s < 15)
            def _():
                pltpu.sync_copy(acc.at[pl.ds(s * _FCH, _FCH)],
                                outR.at[pl.ds(s * _FCH, _FCH)])

            @pl.when(s == 15)
            def _():
                pltpu.sync_copy(acc.at[pl.ds(15 * _FCH, _N - 15 * _FCH)],
                                outR.at[pl.ds(15 * _FCH, _N - 15 * _FCH)])

    return k(msg2, idx2d, zrows)


_BE = 640
_GRID_E = _E // _BE


def _silu(x):
    return x * jax.nn.sigmoid(x)


def _split_store(m2_ref, m):
    m2_ref[0] = m[:, :_HC]
    m2_ref[1] = m[:, _HC:]


def _tc_edge_init(xg, ea, WeX, WeE, be2, W0, b0):
    def body(xg_ref, ea_ref, wx_ref, we_ref, be_ref, w0_ref, b0_ref,
             e_ref, m_ref):
        h = jnp.dot(xg_ref[...], wx_ref[...], preferred_element_type=jnp.float32)
        h = h + jnp.dot(ea_ref[...], we_ref[...], preferred_element_type=jnp.float32)
        h = h + be_ref[...]
        e = _silu(h)
        e_ref[...] = e
        m = jnp.dot(e, w0_ref[...], preferred_element_type=jnp.float32) + b0_ref[...]
        _split_store(m_ref, jnp.maximum(m, 0.0))

    full = lambda shape: pl.BlockSpec(shape, lambda i: (0, 0))
    return pl.pallas_call(
        body,
        grid=(_GRID_E,),
        in_specs=[
            pl.BlockSpec((_BE, 128), lambda i: (i, 0)),
            pl.BlockSpec((_BE, 16), lambda i: (i, 0)),
            full((128, _HP)),
            full((16, _HP)),
            full((1, _HP)),
            full((_HP, _HP)),
            full((1, _HP)),
        ],
        out_specs=[
            pl.BlockSpec((_BE, _HP), lambda i: (i, 0)),
            pl.BlockSpec((2, _BE, _HC), lambda i: (0, i, 0)),
        ],
        out_shape=[
            jax.ShapeDtypeStruct((_E, _HP), jnp.float32),
            jax.ShapeDtypeStruct((2, _E, _HC), jnp.float32),
        ],
        compiler_params=pltpu.CompilerParams(
            dimension_semantics=("parallel",)),
    )(xg, ea, WeX, WeE, be2, W0, b0)


def _tc_layer(agL, agR, ep, mW, mb2, nW, nb2, last):
    def body(agl_ref, agr_ref, ep_ref, mw_ref, mb_ref, nw_ref, nb_ref, *outs):
        ag = jnp.concatenate([agl_ref[...], agr_ref[...]], axis=1)
        d = ag - ep_ref[...]
        h = jnp.dot(d, mw_ref[...], preferred_element_type=jnp.float32) + mb_ref[...]
        h = jnp.maximum(h, 0.0)
        if last:
            e = 2.0 * h
            m_ref, = outs
        else:
            e = _silu(h) + h
            e_ref, m_ref = outs
            e_ref[...] = e
        m = jnp.dot(e, nw_ref[...], preferred_element_type=jnp.float32) + nb_ref[...]
        _split_store(m_ref, jnp.maximum(m, 0.0))

    full = lambda shape: pl.BlockSpec(shape, lambda i: (0, 0))
    out_specs = [pl.BlockSpec((2, _BE, _HC), lambda i: (0, i, 0))]
    out_shape = [jax.ShapeDtypeStruct((2, _E, _HC), jnp.float32)]
    if not last:
        out_specs = [pl.BlockSpec((_BE, _HP), lambda i: (i, 0))] + out_specs
        out_shape = [jax.ShapeDtypeStruct((_E, _HP), jnp.float32)] + out_shape
    return pl.pallas_call(
        body,
        grid=(_GRID_E,),
        in_specs=[
            pl.BlockSpec((_BE, _HC), lambda i: (i, 0)),
            pl.BlockSpec((_BE, _HC), lambda i: (i, 0)),
            pl.BlockSpec((_BE, _HP), lambda i: (i, 0)),
            full((_HP, _HP)),
            full((1, _HP)),
            full((_HP, _HP)),
            full((1, _HP)),
        ],
        out_specs=out_specs,
        out_shape=out_shape,
        compiler_params=pltpu.CompilerParams(
            dimension_semantics=("parallel",)),
    )(agL, agR, ep, mW, mb2, nW, nb2)


_BN = 400
_GRID_N = _N // _BN


def _tc_pool_ffn(nhL, nhR, batch3, W1p, b12, W2, b22, W3, b32):
    def body(nhl_ref, nhr_ref, b_ref, w1_ref, b1_ref, w2_ref, b2_ref,
             w3_ref, b3_ref, out_ref, acc):
        i = pl.program_id(0)
        nh = jnp.concatenate([nhl_ref[...], nhr_ref[...]], axis=1)
        seg = jnp.broadcast_to(b_ref[0], (_G, _BN))
        gids = lax.broadcasted_iota(jnp.int32, (_G, _BN), 0)
        onehot = (seg == gids).astype(jnp.float32)
        part = jnp.dot(onehot, nh, preferred_element_type=jnp.float32)

        @pl.when(i == 0)
        def _():
            acc[...] = jnp.zeros_like(acc)

        acc[...] += part

        @pl.when(i == _GRID_N - 1)
        def _():
            h = jnp.dot(acc[...], w1_ref[...], preferred_element_type=jnp.float32)
            h = _silu(h + b1_ref[...])
            h = jnp.dot(h, w2_ref[...], preferred_element_type=jnp.float32)
            h = _silu(h + b2_ref[...])
            h = jnp.dot(h, w3_ref[...], preferred_element_type=jnp.float32)
            out_ref[...] = h + b3_ref[...]

    full = lambda shape: pl.BlockSpec(shape, lambda i: tuple(0 for _ in shape))
    return pl.pallas_call(
        body,
        grid=(_GRID_N,),
        in_specs=[
            pl.BlockSpec((_BN, _HC), lambda i: (i, 0)),
            pl.BlockSpec((_BN, _HC), lambda i: (i, 0)),
            pl.BlockSpec((1, 1, _BN), lambda i: (i, 0, 0)),
            full((_HP, 300)),
            full((1, 300)),
            full((300, 300)),
            full((1, 300)),
            full((300, 1)),
            full((1, 1)),
        ],
        out_specs=pl.BlockSpec((_G, 1), lambda i: (0, 0)),
        out_shape=jax.ShapeDtypeStruct((_G, 1), jnp.float32),
        scratch_shapes=[pltpu.VMEM((_G, _HP), jnp.float32)],
        compiler_params=pltpu.CompilerParams(
            dimension_semantics=("arbitrary",)),
    )(nhL, nhR, batch3, W1p, b12, W2, b22, W3, b32)


def kernel(x, edge_index, edge_attr, batch, We, be, linW, linb, mlpW, mlpb,
           n2W, n2b, W1, b1, W2, b2, W3, b3):
    f32 = jnp.float32
    row = edge_index[0].astype(jnp.int32)
    col = edge_index[1].astype(jnp.int32)
    row_sw = row.reshape(_E // 2, 2)[:, ::-1].reshape(_NCHUNK, _CH)
    col_sw = col.reshape(_E // 2, 2)[:, ::-1].reshape(_NCHUNK_S, _SCH)
    row = row.reshape(_NCHUNK, _CH)
    col = col.reshape(_NCHUNK_S, _SCH)

    pad_h = lambda w: jnp.pad(w, ((0, _HP - w.shape[0]), (0, _HP - w.shape[1])))
    pad_b = lambda v: jnp.pad(v, (0, _HP - v.shape[0])).reshape(1, _HP)

    WeX = jnp.pad(We[:128], ((0, 0), (0, _HP - 300)))
    WeE = jnp.pad(We[128:], ((0, 0), (0, _HP - 300)))
    be2 = pad_b(be)
    linWp = [pad_h(linW[l]) for l in range(3)]
    linbp = [pad_b(linb[l]) for l in range(3)]
    mlpWp = [pad_h(mlpW[l]) for l in range(3)]
    mlpbp = [pad_b(mlpb[l]) for l in range(3)]
    n2Wp = pad_h(n2W)
    n2bp = pad_b(n2b)
    W1p = jnp.pad(W1, ((0, _HP - 300), (0, 0)))
    b12 = b1.reshape(1, 300)
    b22 = b2.reshape(1, 300)
    b32 = b3.reshape(1, 1)
    batch3 = batch.astype(jnp.int32).reshape(_GRID_N, 1, _BN)
    zrows = jnp.zeros((_ZCH, _HC), f32)

    xg, = _sc_gather([x], row, [128])
    e0, msg0 = _tc_edge_init(xg, edge_attr, WeX, WeE, be2, linWp[0], linbp[0])

    a0L, a0R = _sc_scatter_add(msg0, col, zrows)
    ag0L, ag0R = _sc_gather([a0L, a0R], row_sw, [_HC, _HC])
    e1s, msg1s = _tc_layer(ag0L, ag0R, e0, mlpWp[0], mlpbp[0],
                           linWp[1], linbp[1], False)

    a1L, a1R = _sc_scatter_add(msg1s, col_sw, zrows)
    ag1L, ag1R = _sc_gather([a1L, a1R], row, [_HC, _HC])
    e2, msg2 = _tc_layer(ag1L, ag1R, e1s, mlpWp[1], mlpbp[1],
                         linWp[2], linbp[2], False)

    a2L, a2R = _sc_scatter_add(msg2, col, zrows)
    ag2L, ag2R = _sc_gather([a2L, a2R], row_sw, [_HC, _HC])
    msgf, = _tc_layer(ag2L, ag2R, e2, mlpWp[2], mlpbp[2], n2Wp, n2bp, True)

    nhL, nhR = _sc_scatter_add(msgf, col_sw, zrows)
    return _tc_pool_ffn(nhL, nhR, batch3, W1p, b12, W2, b22, W3, b32)
